# initial kernel scaffold (unmeasured)
import jax
import jax.numpy as jnp
from jax import lax
from jax.experimental import pallas as pl
from jax.experimental.pallas import tpu as pltpu

N_DEV = 4
M = 2048
D = 2048
H = D // 2


def kernel(partial, gamma):
    x = partial.reshape(N_DEV * M, D)
    g = gamma.reshape(1, D)

    def body(x_hbm, g_ref, o_ref,
             stage_p, stage_m, send_p, send_m, recv_p, recv_m,
             load_sems, send_sems_p, recv_sems_p, send_sems_m, recv_sems_m):
        z = lax.axis_index("z")
        mx = lax.axis_index("x")
        my = lax.axis_index("y")
        right = lax.rem(z + 1, N_DEV)
        left = lax.rem(z + N_DEV - 1, N_DEV)

        barrier = pltpu.get_barrier_semaphore()
        pl.semaphore_signal(barrier, inc=1, device_id=(mx, my, left),
                            device_id_type=pl.DeviceIdType.MESH)
        pl.semaphore_signal(barrier, inc=1, device_id=(mx, my, right),
                            device_id_type=pl.DeviceIdType.MESH)
        pl.semaphore_wait(barrier, 2)

        def load(idx, col0, dst, sem):
            cp = pltpu.make_async_copy(
                x_hbm.at[pl.ds(idx * M, M), pl.ds(col0, H)], dst, sem)
            cp.start()
            return cp

        for s in range(N_DEV - 1):
            idx_p = lax.rem(z - 1 - s + 2 * N_DEV, N_DEV)
            idx_m = lax.rem(z + 1 + s, N_DEV)
            cp_p = load(idx_p, 0, stage_p, load_sems.at[0])
            cp_m = load(idx_m, H, stage_m, load_sems.at[1])
            cp_p.wait()
            cp_m.wait()
            if s == 0:
                send_p[...] = stage_p[...].astype(jnp.bfloat16)
                send_m[...] = stage_m[...].astype(jnp.bfloat16)
            else:
                send_p[...] = (recv_p[s - 1].astype(jnp.float32)
                               + stage_p[...]).astype(jnp.bfloat16)
                send_m[...] = (recv_m[s - 1].astype(jnp.float32)
                               + stage_m[...]).astype(jnp.bfloat16)
            rp = pltpu.make_async_remote_copy(
                src_ref=send_p, dst_ref=recv_p.at[s],
                send_sem=send_sems_p.at[s], recv_sem=recv_sems_p.at[s],
                device_id=(mx, my, right),
                device_id_type=pl.DeviceIdType.MESH)
            rm = pltpu.make_async_remote_copy(
                src_ref=send_m, dst_ref=recv_m.at[s],
                send_sem=send_sems_m.at[s], recv_sem=recv_sems_m.at[s],
                device_id=(mx, my, left),
                device_id_type=pl.DeviceIdType.MESH)
            rp.start()
            rm.start()
            rp.wait()
            rm.wait()

        cp_p = load(z, 0, stage_p, load_sems.at[0])
        cp_m = load(z, H, stage_m, load_sems.at[1])
        cp_p.wait()
        cp_m.wait()
        y_a = recv_p[N_DEV - 2].astype(jnp.float32) + stage_p[...]
        y_b = recv_m[N_DEV - 2].astype(jnp.float32) + stage_m[...]
        ss = (jnp.sum(y_a * y_a, axis=1, keepdims=True)
              + jnp.sum(y_b * y_b, axis=1, keepdims=True))
        inv = lax.rsqrt(ss / D + 1e-6)
        o_ref[:, 0:H] = y_a * inv * g_ref[:, 0:H]
        o_ref[:, H:D] = y_b * inv * g_ref[:, H:D]

    return pl.pallas_call(
        body,
        out_shape=jax.ShapeDtypeStruct((M, D), jnp.float32),
        in_specs=[pl.BlockSpec(memory_space=pltpu.ANY),
                  pl.BlockSpec(memory_space=pltpu.VMEM)],
        out_specs=pl.BlockSpec(memory_space=pltpu.VMEM),
        scratch_shapes=[
            pltpu.VMEM((M, H), jnp.float32),
            pltpu.VMEM((M, H), jnp.float32),
            pltpu.VMEM((M, H), jnp.bfloat16),
            pltpu.VMEM((M, H), jnp.bfloat16),
            pltpu.VMEM((N_DEV - 1, M, H), jnp.bfloat16),
            pltpu.VMEM((N_DEV - 1, M, H), jnp.bfloat16),
            pltpu.SemaphoreType.DMA((2,)),
            pltpu.SemaphoreType.DMA((N_DEV - 1,)),
            pltpu.SemaphoreType.DMA((N_DEV - 1,)),
            pltpu.SemaphoreType.DMA((N_DEV - 1,)),
            pltpu.SemaphoreType.DMA((N_DEV - 1,)),
        ],
        compiler_params=pltpu.CompilerParams(collective_id=0),
    )(x, g)


# baseline (device time: 327527 ns/iter reference)
import jax
import jax.numpy as jnp
from jax import lax
from jax.experimental import pallas as pl
from jax.experimental.pallas import tpu as pltpu

N_DEV = 4
M = 2048
D = 2048
H = D // 2
BLK = 512
F32 = jnp.float32
BF16 = jnp.bfloat16


def kernel(partial, gamma):
    x = partial.reshape(N_DEV * M, D)
    g = gamma.reshape(1, D)

    def body(x_hbm, g_ref, o_hbm,
             stage_p, stage_m, send_p, send_m, recv_p, recv_m,
             load_sems, out_sems,
             send_sems_p, recv_sems_p, send_sems_m, recv_sems_m):
        z = lax.axis_index("z")
        mx = lax.axis_index("x")
        my = lax.axis_index("y")
        right = lax.rem(z + 1, N_DEV)
        left = lax.rem(z + N_DEV - 1, N_DEV)

        barrier = pltpu.get_barrier_semaphore()
        pl.semaphore_signal(barrier, inc=1, device_id=(mx, my, left),
                            device_id_type=pl.DeviceIdType.MESH)
        pl.semaphore_signal(barrier, inc=1, device_id=(mx, my, right),
                            device_id_type=pl.DeviceIdType.MESH)
        pl.semaphore_wait(barrier, 2)

        def load(idx, col0, dst, sem):
            cp = pltpu.make_async_copy(
                x_hbm.at[pl.ds(idx * M, M), pl.ds(col0, H)], dst, sem)
            cp.start()
            return cp

        for s in range(N_DEV - 1):
            idx_p = lax.rem(z - 1 - s + 2 * N_DEV, N_DEV)
            idx_m = lax.rem(z + 1 + s, N_DEV)
            cp_p = load(idx_p, 0, stage_p, load_sems.at[0])
            cp_m = load(idx_m, H, stage_m, load_sems.at[1])
            cp_p.wait()
            cp_m.wait()
            for r in range(0, M, BLK):
                rs = slice(r, r + BLK)
                if s == 0:
                    send_p[rs, :] = stage_p[rs, :].astype(BF16)
                    send_m[rs, :] = stage_m[rs, :].astype(BF16)
                else:
                    send_p[rs, :] = (recv_p[s - 1, rs, :].astype(F32)
                                     + stage_p[rs, :]).astype(BF16)
                    send_m[rs, :] = (recv_m[s - 1, rs, :].astype(F32)
                                     + stage_m[rs, :]).astype(BF16)
            rp = pltpu.make_async_remote_copy(
                src_ref=send_p, dst_ref=recv_p.at[s],
                send_sem=send_sems_p.at[s], recv_sem=recv_sems_p.at[s],
                device_id=(mx, my, right),
                device_id_type=pl.DeviceIdType.MESH)
            rm = pltpu.make_async_remote_copy(
                src_ref=send_m, dst_ref=recv_m.at[s],
                send_sem=send_sems_m.at[s], recv_sem=recv_sems_m.at[s],
                device_id=(mx, my, left),
                device_id_type=pl.DeviceIdType.MESH)
            rp.start()
            rm.start()
            rp.wait()
            rm.wait()

        cp_p = load(z, 0, stage_p, load_sems.at[0])
        cp_m = load(z, H, stage_m, load_sems.at[1])
        cp_p.wait()
        cp_m.wait()
        ss_parts = []
        for r in range(0, M, BLK):
            rs = slice(r, r + BLK)
            a = recv_p[N_DEV - 2, rs, :].astype(F32) + stage_p[rs, :]
            b = recv_m[N_DEV - 2, rs, :].astype(F32) + stage_m[rs, :]
            stage_p[rs, :] = a
            stage_m[rs, :] = b
            ss_parts.append(jnp.sum(a * a, axis=1, keepdims=True)
                            + jnp.sum(b * b, axis=1, keepdims=True))
        for i, r in enumerate(range(0, M, BLK)):
            rs = slice(r, r + BLK)
            inv = lax.rsqrt(ss_parts[i] / D + 1e-6)
            stage_p[rs, :] = stage_p[rs, :] * inv * g_ref[:, 0:H]
            stage_m[rs, :] = stage_m[rs, :] * inv * g_ref[:, H:D]
        op = pltpu.make_async_copy(
            stage_p, o_hbm.at[:, pl.ds(0, H)], out_sems.at[0])
        om = pltpu.make_async_copy(
            stage_m, o_hbm.at[:, pl.ds(H, H)], out_sems.at[1])
        op.start()
        om.start()
        op.wait()
        om.wait()

    return pl.pallas_call(
        body,
        out_shape=jax.ShapeDtypeStruct((M, D), jnp.float32),
        in_specs=[pl.BlockSpec(memory_space=pl.ANY),
                  pl.BlockSpec(memory_space=pltpu.MemorySpace.VMEM)],
        out_specs=pl.BlockSpec(memory_space=pl.ANY),
        scratch_shapes=[
            pltpu.VMEM((M, H), jnp.float32),
            pltpu.VMEM((M, H), jnp.float32),
            pltpu.VMEM((M, H), jnp.bfloat16),
            pltpu.VMEM((M, H), jnp.bfloat16),
            pltpu.VMEM((N_DEV - 1, M, H), jnp.bfloat16),
            pltpu.VMEM((N_DEV - 1, M, H), jnp.bfloat16),
            pltpu.SemaphoreType.DMA((2,)),
            pltpu.SemaphoreType.DMA((2,)),
            pltpu.SemaphoreType.DMA((N_DEV - 1,)),
            pltpu.SemaphoreType.DMA((N_DEV - 1,)),
            pltpu.SemaphoreType.DMA((N_DEV - 1,)),
            pltpu.SemaphoreType.DMA((N_DEV - 1,)),
        ],
        compiler_params=pltpu.CompilerParams(
            collective_id=0, vmem_limit_bytes=100 * 1024 * 1024),
    )(x, g)


# device time: 240111 ns/iter; 1.3641x vs baseline; 1.3641x over previous
import jax
import jax.numpy as jnp
from jax import lax
from jax.experimental import pallas as pl
from jax.experimental.pallas import tpu as pltpu

N_DEV = 4
M = 2048
D = 2048
HW = D // 2
QW = HW // 2
BLK = 512
F32 = jnp.float32
BF16 = jnp.bfloat16


def kernel(partial, gamma):
    x = partial.reshape(N_DEV * M, D)
    mx_out = lax.axis_index("x")
    g_own = lax.dynamic_slice(gamma, (mx_out * HW,), (HW,))
    g2 = g_own.reshape(1, HW)

    def body(x_hbm, g_ref, o_hbm,
             stage_p, stage_m, send_p, send_m, recv_p, recv_m,
             ybuf, yx_send, yx_recv, ss_send, ss_recv,
             load_sems, out_sems,
             send_sems_p, recv_sems_p, send_sems_m, recv_sems_m,
             ss_sems, yx_sems):
        z = lax.axis_index("z")
        mx = lax.axis_index("x")
        my = lax.axis_index("y")
        right = lax.rem(z + 1, N_DEV)
        left = lax.rem(z + N_DEV - 1, N_DEV)
        px = 1 - mx
        base = mx * HW

        barrier = pltpu.get_barrier_semaphore()
        for dev in [(mx, my, left), (mx, my, right), (px, my, z)]:
            pl.semaphore_signal(barrier, inc=1, device_id=dev,
                                device_id_type=pl.DeviceIdType.MESH)
        pl.semaphore_wait(barrier, 3)

        def load(idx, col0, dst, sem):
            cp = pltpu.make_async_copy(
                x_hbm.at[pl.ds(idx * M, M), pl.ds(col0, QW)], dst, sem)
            cp.start()
            return cp

        for s in range(N_DEV - 1):
            idx_p = lax.rem(z - 1 - s + 2 * N_DEV, N_DEV)
            idx_m = lax.rem(z + 1 + s, N_DEV)
            cp_p = load(idx_p, base, stage_p, load_sems.at[0])
            cp_m = load(idx_m, base + QW, stage_m, load_sems.at[1])
            cp_p.wait()
            cp_m.wait()
            for r in range(0, M, BLK):
                rs = slice(r, r + BLK)
                if s == 0:
                    send_p[rs, :] = stage_p[rs, :].astype(BF16)
                    send_m[rs, :] = stage_m[rs, :].astype(BF16)
                else:
                    send_p[rs, :] = (recv_p[s - 1, rs, :].astype(F32)
                                     + stage_p[rs, :]).astype(BF16)
                    send_m[rs, :] = (recv_m[s - 1, rs, :].astype(F32)
                                     + stage_m[rs, :]).astype(BF16)
            rp = pltpu.make_async_remote_copy(
                src_ref=send_p, dst_ref=recv_p.at[s],
                send_sem=send_sems_p.at[s], recv_sem=recv_sems_p.at[s],
                device_id=(mx, my, right),
                device_id_type=pl.DeviceIdType.MESH)
            rm = pltpu.make_async_remote_copy(
                src_ref=send_m, dst_ref=recv_m.at[s],
                send_sem=send_sems_m.at[s], recv_sem=recv_sems_m.at[s],
                device_id=(mx, my, left),
                device_id_type=pl.DeviceIdType.MESH)
            rp.start()
            rm.start()
            rp.wait()
            rm.wait()

        cp_p = load(z, base, stage_p, load_sems.at[0])
        cp_m = load(z, base + QW, stage_m, load_sems.at[1])
        cp_p.wait()
        cp_m.wait()
        for r in range(0, M, BLK):
            rs = slice(r, r + BLK)
            a = recv_p[N_DEV - 2, rs, :].astype(F32) + stage_p[rs, :]
            b = recv_m[N_DEV - 2, rs, :].astype(F32) + stage_m[rs, :]
            ybuf[rs, 0:QW] = a
            ybuf[rs, QW:HW] = b
            ss_send[rs, :] = (jnp.sum(a * a, axis=1, keepdims=True)
                              + jnp.sum(b * b, axis=1, keepdims=True))

        r_ss = pltpu.make_async_remote_copy(
            src_ref=ss_send, dst_ref=ss_recv,
            send_sem=ss_sems.at[0], recv_sem=ss_sems.at[1],
            device_id=(px, my, z), device_id_type=pl.DeviceIdType.MESH)
        r_ss.start()
        r_ss.wait()

        for r in range(0, M, BLK):
            rs = slice(r, r + BLK)
            inv = lax.rsqrt((ss_send[rs, :] + ss_recv[rs, :]) / D + 1e-6)
            yn = ybuf[rs, 0:HW] * inv * g_ref[...]
            ybuf[rs, 0:HW] = yn
            yx_send[rs, :] = yn.astype(BF16)

        r_yx = pltpu.make_async_remote_copy(
            src_ref=yx_send, dst_ref=yx_recv,
            send_sem=yx_sems.at[0], recv_sem=yx_sems.at[1],
            device_id=(px, my, z), device_id_type=pl.DeviceIdType.MESH)
        r_yx.start()
        o_own = pltpu.make_async_copy(
            ybuf.at[:, pl.ds(0, HW)], o_hbm.at[:, pl.ds(base, HW)],
            out_sems.at[0])
        o_own.start()
        r_yx.wait()
        for r in range(0, M, BLK):
            rs = slice(r, r + BLK)
            ybuf[rs, HW:D] = yx_recv[rs, :].astype(F32)
        o_part = pltpu.make_async_copy(
            ybuf.at[:, pl.ds(HW, HW)], o_hbm.at[:, pl.ds(px * HW, HW)],
            out_sems.at[1])
        o_part.start()
        o_own.wait()
        o_part.wait()

    return pl.pallas_call(
        body,
        out_shape=jax.ShapeDtypeStruct((M, D), jnp.float32),
        in_specs=[pl.BlockSpec(memory_space=pl.ANY),
                  pl.BlockSpec(memory_space=pltpu.MemorySpace.VMEM)],
        out_specs=pl.BlockSpec(memory_space=pl.ANY),
        scratch_shapes=[
            pltpu.VMEM((M, QW), jnp.float32),
            pltpu.VMEM((M, QW), jnp.float32),
            pltpu.VMEM((M, QW), jnp.bfloat16),
            pltpu.VMEM((M, QW), jnp.bfloat16),
            pltpu.VMEM((N_DEV - 1, M, QW), jnp.bfloat16),
            pltpu.VMEM((N_DEV - 1, M, QW), jnp.bfloat16),
            pltpu.VMEM((M, D), jnp.float32),
            pltpu.VMEM((M, HW), jnp.bfloat16),
            pltpu.VMEM((M, HW), jnp.bfloat16),
            pltpu.VMEM((M, 1), jnp.float32),
            pltpu.VMEM((M, 1), jnp.float32),
            pltpu.SemaphoreType.DMA((2,)),
            pltpu.SemaphoreType.DMA((2,)),
            pltpu.SemaphoreType.DMA((N_DEV - 1,)),
            pltpu.SemaphoreType.DMA((N_DEV - 1,)),
            pltpu.SemaphoreType.DMA((N_DEV - 1,)),
            pltpu.SemaphoreType.DMA((N_DEV - 1,)),
            pltpu.SemaphoreType.DMA((2,)),
            pltpu.SemaphoreType.DMA((2,)),
        ],
        compiler_params=pltpu.CompilerParams(
            collective_id=0, vmem_limit_bytes=100 * 1024 * 1024),
    )(x, g2)


# device time: 226246 ns/iter; 1.4477x vs baseline; 1.0613x over previous
import jax
import jax.numpy as jnp
from jax import lax
from jax.experimental import pallas as pl
from jax.experimental.pallas import tpu as pltpu

N_DEV = 4
M = 2048
D = 2048
HW = D // 2
QW = HW // 2
BLK = 512
RB = 1024
F32 = jnp.float32
BF16 = jnp.bfloat16


def kernel(partial, gamma):
    x = partial.reshape(N_DEV * M, D)
    mx_out = lax.axis_index("x")
    g_own = lax.dynamic_slice(gamma, (mx_out * HW,), (HW,))
    g2 = g_own.reshape(1, HW)

    def body(x_hbm, g_ref, o_hbm,
             stage_p, stage_m, send_p, send_m, recv_p, recv_m,
             ybuf, yx_send, yx_recv, ss_send, ss_recv,
             load_sems_p, load_sems_m, out_sems,
             send_sems_p, recv_sems_p, send_sems_m, recv_sems_m,
             ss_sems, yx_send_sems, yx_recv_sems):
        z = lax.axis_index("z")
        mx = lax.axis_index("x")
        my = lax.axis_index("y")
        right = lax.rem(z + 1, N_DEV)
        left = lax.rem(z + N_DEV - 1, N_DEV)
        px = 1 - mx
        base = mx * HW

        def load(idx, col0, dst, sem):
            cp = pltpu.make_async_copy(
                x_hbm.at[pl.ds(idx * M, M), pl.ds(col0, QW)], dst, sem)
            cp.start()
            return cp

        def idx_pair(s):
            if s == N_DEV - 1:
                return z, z
            return (lax.rem(z - 1 - s + 2 * N_DEV, N_DEV),
                    lax.rem(z + 1 + s, N_DEV))

        ip0, im0 = idx_pair(0)
        cps = {0: (load(ip0, base, stage_p.at[0], load_sems_p.at[0]),
                   load(im0, base + QW, stage_m.at[0], load_sems_m.at[0]))}

        barrier = pltpu.get_barrier_semaphore()
        for dev in [(mx, my, left), (mx, my, right), (px, my, z)]:
            pl.semaphore_signal(barrier, inc=1, device_id=dev,
                                device_id_type=pl.DeviceIdType.MESH)
        pl.semaphore_wait(barrier, 3)

        for s in range(N_DEV - 1):
            slot = s % 2
            nxt = (s + 1) % 2
            cp_p, cp_m = cps.pop(s)
            cp_p.wait()
            cp_m.wait()
            ipn, imn = idx_pair(s + 1)
            cps[s + 1] = (load(ipn, base, stage_p.at[nxt],
                               load_sems_p.at[nxt]),
                          load(imn, base + QW, stage_m.at[nxt],
                               load_sems_m.at[nxt]))
            for r in range(0, M, BLK):
                rs = slice(r, r + BLK)
                if s == 0:
                    send_p[rs, :] = stage_p[slot, rs, :].astype(BF16)
                    send_m[rs, :] = stage_m[slot, rs, :].astype(BF16)
                else:
                    send_p[rs, :] = (recv_p[s - 1, rs, :].astype(F32)
                                     + stage_p[slot, rs, :]).astype(BF16)
                    send_m[rs, :] = (recv_m[s - 1, rs, :].astype(F32)
                                     + stage_m[slot, rs, :]).astype(BF16)
            rp = pltpu.make_async_remote_copy(
                src_ref=send_p, dst_ref=recv_p.at[s],
                send_sem=send_sems_p.at[s], recv_sem=recv_sems_p.at[s],
                device_id=(mx, my, right),
                device_id_type=pl.DeviceIdType.MESH)
            rm = pltpu.make_async_remote_copy(
                src_ref=send_m, dst_ref=recv_m.at[s],
                send_sem=send_sems_m.at[s], recv_sem=recv_sems_m.at[s],
                device_id=(mx, my, left),
                device_id_type=pl.DeviceIdType.MESH)
            rp.start()
            rm.start()
            rp.wait()
            rm.wait()

        fslot = (N_DEV - 1) % 2
        cp_p, cp_m = cps.pop(N_DEV - 1)
        cp_p.wait()
        cp_m.wait()
        for r in range(0, M, BLK):
            rs = slice(r, r + BLK)
            a = recv_p[N_DEV - 2, rs, :].astype(F32) + stage_p[fslot, rs, :]
            b = recv_m[N_DEV - 2, rs, :].astype(F32) + stage_m[fslot, rs, :]
            ybuf[rs, 0:QW] = a
            ybuf[rs, QW:HW] = b
            ss_send[rs, :] = (jnp.sum(a * a, axis=1, keepdims=True)
                              + jnp.sum(b * b, axis=1, keepdims=True))

        r_ss = pltpu.make_async_remote_copy(
            src_ref=ss_send, dst_ref=ss_recv,
            send_sem=ss_sems.at[0], recv_sem=ss_sems.at[1],
            device_id=(px, my, z), device_id_type=pl.DeviceIdType.MESH)
        r_ss.start()
        r_ss.wait()

        yx_rdmas = []
        o_owns = []
        for bi, rb in enumerate(range(0, M, RB)):
            for r in range(rb, rb + RB, BLK):
                rs = slice(r, r + BLK)
                inv = lax.rsqrt((ss_send[rs, :] + ss_recv[rs, :]) / D + 1e-6)
                yn = ybuf[rs, 0:HW] * inv * g_ref[...]
                ybuf[rs, 0:HW] = yn
                yx_send[rs, :] = yn.astype(BF16)
            r_yx = pltpu.make_async_remote_copy(
                src_ref=yx_send.at[pl.ds(rb, RB), :],
                dst_ref=yx_recv.at[pl.ds(rb, RB), :],
                send_sem=yx_send_sems.at[bi], recv_sem=yx_recv_sems.at[bi],
                device_id=(px, my, z), device_id_type=pl.DeviceIdType.MESH)
            r_yx.start()
            yx_rdmas.append(r_yx)
            o_own = pltpu.make_async_copy(
                ybuf.at[pl.ds(rb, RB), pl.ds(0, HW)],
                o_hbm.at[pl.ds(rb, RB), pl.ds(base, HW)],
                out_sems.at[bi])
            o_own.start()
            o_owns.append(o_own)

        o_parts = []
        for bi, rb in enumerate(range(0, M, RB)):
            yx_rdmas[bi].wait()
            for r in range(rb, rb + RB, BLK):
                rs = slice(r, r + BLK)
                ybuf[rs, HW:D] = yx_recv[rs, :].astype(F32)
            o_part = pltpu.make_async_copy(
                ybuf.at[pl.ds(rb, RB), pl.ds(HW, HW)],
                o_hbm.at[pl.ds(rb, RB), pl.ds(px * HW, HW)],
                out_sems.at[2 + bi])
            o_part.start()
            o_parts.append(o_part)
        for cp in o_owns + o_parts:
            cp.wait()

    return pl.pallas_call(
        body,
        out_shape=jax.ShapeDtypeStruct((M, D), jnp.float32),
        in_specs=[pl.BlockSpec(memory_space=pl.ANY),
                  pl.BlockSpec(memory_space=pltpu.MemorySpace.VMEM)],
        out_specs=pl.BlockSpec(memory_space=pl.ANY),
        scratch_shapes=[
            pltpu.VMEM((2, M, QW), jnp.float32),
            pltpu.VMEM((2, M, QW), jnp.float32),
            pltpu.VMEM((M, QW), jnp.bfloat16),
            pltpu.VMEM((M, QW), jnp.bfloat16),
            pltpu.VMEM((N_DEV - 1, M, QW), jnp.bfloat16),
            pltpu.VMEM((N_DEV - 1, M, QW), jnp.bfloat16),
            pltpu.VMEM((M, D), jnp.float32),
            pltpu.VMEM((M, HW), jnp.bfloat16),
            pltpu.VMEM((M, HW), jnp.bfloat16),
            pltpu.VMEM((M, 1), jnp.float32),
            pltpu.VMEM((M, 1), jnp.float32),
            pltpu.SemaphoreType.DMA((2,)),
            pltpu.SemaphoreType.DMA((2,)),
            pltpu.SemaphoreType.DMA((4,)),
            pltpu.SemaphoreType.DMA((N_DEV - 1,)),
            pltpu.SemaphoreType.DMA((N_DEV - 1,)),
            pltpu.SemaphoreType.DMA((N_DEV - 1,)),
            pltpu.SemaphoreType.DMA((N_DEV - 1,)),
            pltpu.SemaphoreType.DMA((2,)),
            pltpu.SemaphoreType.DMA((2,)),
            pltpu.SemaphoreType.DMA((2,)),
        ],
        compiler_params=pltpu.CompilerParams(
            collective_id=0, vmem_limit_bytes=100 * 1024 * 1024),
    )(x, g2)


# device time: 162130 ns/iter; 2.0202x vs baseline; 1.3955x over previous
import jax
import jax.numpy as jnp
from jax import lax
from jax.experimental import pallas as pl
from jax.experimental.pallas import tpu as pltpu

N_DEV = 4
M = 2048
D = 2048
HW = D // 2
PW = HW // N_DEV
OW = PW // 2
BLK = 512
F32 = jnp.float32
BF16 = jnp.bfloat16


def kernel(partial, gamma):
    x = partial.reshape(N_DEV * M, D)
    mx_out = lax.axis_index("x")
    g_own = lax.dynamic_slice(gamma, (mx_out * HW,), (HW,))
    g_par = lax.dynamic_slice(gamma, ((1 - mx_out) * HW,), (HW,))
    g2 = jnp.concatenate([g_own, g_par]).reshape(1, D)

    def body(x_hbm, g_ref, o_hbm,
             stage_p, stage_m, send_p, send_m, recv_p, recv_m,
             pack_own, buf_y, buf_x, ybuf,
             load_sems_p, load_sems_m, copy_sem, out_sems,
             send_sems_p, recv_sems_p, send_sems_m, recv_sems_m,
             y_send_sems, y_recv_sems, x_send_sems, x_recv_sems):
        z = lax.axis_index("z")
        mx = lax.axis_index("x")
        my = lax.axis_index("y")
        right = lax.rem(z + 1, N_DEV)
        left = lax.rem(z + N_DEV - 1, N_DEV)
        px = 1 - mx
        cbase = mx * HW + my * PW

        def load(idx, col0, dst, sem):
            cp = pltpu.make_async_copy(
                x_hbm.at[pl.ds(idx * M, M), pl.ds(col0, OW)], dst, sem)
            cp.start()
            return cp

        def idx_pair(s):
            if s == N_DEV - 1:
                return z, z
            return (lax.rem(z - 1 - s + 2 * N_DEV, N_DEV),
                    lax.rem(z + 1 + s, N_DEV))

        ip0, im0 = idx_pair(0)
        cps = {0: (load(ip0, cbase, stage_p.at[0], load_sems_p.at[0]),
                   load(im0, cbase + OW, stage_m.at[0], load_sems_m.at[0]))}

        barrier = pltpu.get_barrier_semaphore()
        for dev in [(mx, my, left), (mx, my, right), (px, my, z)]:
            pl.semaphore_signal(barrier, inc=1, device_id=dev,
                                device_id_type=pl.DeviceIdType.MESH)
        for off in range(1, N_DEV):
            pl.semaphore_signal(barrier, inc=1,
                                device_id=(mx, lax.rem(my + off, N_DEV), z),
                                device_id_type=pl.DeviceIdType.MESH)
        pl.semaphore_wait(barrier, 6)

        for s in range(N_DEV - 1):
            slot = s % 2
            nxt = (s + 1) % 2
            cp_p, cp_m = cps.pop(s)
            cp_p.wait()
            cp_m.wait()
            ipn, imn = idx_pair(s + 1)
            cps[s + 1] = (load(ipn, cbase, stage_p.at[nxt],
                               load_sems_p.at[nxt]),
                          load(imn, cbase + OW, stage_m.at[nxt],
                               load_sems_m.at[nxt]))
            for r in range(0, M, BLK):
                rs = slice(r, r + BLK)
                if s == 0:
                    send_p[rs, :] = stage_p[slot, rs, :].astype(BF16)
                    send_m[rs, :] = stage_m[slot, rs, :].astype(BF16)
                else:
                    send_p[rs, :] = (recv_p[s - 1, rs, :].astype(F32)
                                     + stage_p[slot, rs, :]).astype(BF16)
                    send_m[rs, :] = (recv_m[s - 1, rs, :].astype(F32)
                                     + stage_m[slot, rs, :]).astype(BF16)
            rp = pltpu.make_async_remote_copy(
                src_ref=send_p, dst_ref=recv_p.at[s],
                send_sem=send_sems_p.at[s], recv_sem=recv_sems_p.at[s],
                device_id=(mx, my, right),
                device_id_type=pl.DeviceIdType.MESH)
            rm = pltpu.make_async_remote_copy(
                src_ref=send_m, dst_ref=recv_m.at[s],
                send_sem=send_sems_m.at[s], recv_sem=recv_sems_m.at[s],
                device_id=(mx, my, left),
                device_id_type=pl.DeviceIdType.MESH)
            rp.start()
            rm.start()
            rp.wait()
            rm.wait()

        fslot = (N_DEV - 1) % 2
        cp_p, cp_m = cps.pop(N_DEV - 1)
        cp_p.wait()
        cp_m.wait()
        for r in range(0, M, BLK):
            rs = slice(r, r + BLK)
            pack_own[rs, 0:OW] = (recv_p[N_DEV - 2, rs, :].astype(F32)
                                  + stage_p[fslot, rs, :]).astype(BF16)
            pack_own[rs, OW:PW] = (recv_m[N_DEV - 2, rs, :].astype(F32)
                                   + stage_m[fslot, rs, :]).astype(BF16)

        y_rdmas = []
        for off in range(1, N_DEV):
            r_y = pltpu.make_async_remote_copy(
                src_ref=pack_own, dst_ref=buf_y.at[my],
                send_sem=y_send_sems.at[off - 1],
                recv_sem=y_recv_sems.at[off],
                device_id=(mx, lax.rem(my + off, N_DEV), z),
                device_id_type=pl.DeviceIdType.MESH)
            r_y.start()
            y_rdmas.append(r_y)
        cp_self = pltpu.make_async_copy(pack_own, buf_y.at[my], copy_sem)
        cp_self.start()
        cp_self.wait()
        for off in range(1, N_DEV):
            pltpu.make_async_remote_copy(
                src_ref=pack_own, dst_ref=buf_y.at[0],
                send_sem=y_send_sems.at[off - 1],
                recv_sem=y_recv_sems.at[off],
                device_id=(mx, my, z),
                device_id_type=pl.DeviceIdType.MESH).wait_recv()

        x_rdmas = []
        for sl in range(N_DEV):
            r_x = pltpu.make_async_remote_copy(
                src_ref=buf_y.at[sl], dst_ref=buf_x.at[sl],
                send_sem=x_send_sems.at[sl], recv_sem=x_recv_sems.at[sl],
                device_id=(px, my, z), device_id_type=pl.DeviceIdType.MESH)
            r_x.start()
            x_rdmas.append(r_x)
        for r_x in x_rdmas:
            r_x.wait_recv()
        for r_y in y_rdmas:
            r_y.wait_send()
        for r_x in x_rdmas:
            r_x.wait_send()

        for r in range(0, M, BLK):
            rs = slice(r, r + BLK)
            ssq = jnp.zeros((BLK, 1), F32)
            for sl in range(N_DEV):
                v = buf_y[sl, rs, :].astype(F32)
                ybuf[rs, sl * PW:(sl + 1) * PW] = v
                ssq = ssq + jnp.sum(v * v, axis=1, keepdims=True)
            for sl in range(N_DEV):
                v = buf_x[sl, rs, :].astype(F32)
                ybuf[rs, HW + sl * PW:HW + (sl + 1) * PW] = v
                ssq = ssq + jnp.sum(v * v, axis=1, keepdims=True)
            inv = lax.rsqrt(ssq / D + 1e-6)
            ybuf[rs, 0:HW] = ybuf[rs, 0:HW] * inv * g_ref[:, 0:HW]
            ybuf[rs, HW:D] = ybuf[rs, HW:D] * inv * g_ref[:, HW:D]
        o_own = pltpu.make_async_copy(
            ybuf.at[:, pl.ds(0, HW)], o_hbm.at[:, pl.ds(mx * HW, HW)],
            out_sems.at[0])
        o_par = pltpu.make_async_copy(
            ybuf.at[:, pl.ds(HW, HW)], o_hbm.at[:, pl.ds(px * HW, HW)],
            out_sems.at[1])
        o_own.start()
        o_par.start()
        o_own.wait()
        o_par.wait()

    return pl.pallas_call(
        body,
        out_shape=jax.ShapeDtypeStruct((M, D), jnp.float32),
        in_specs=[pl.BlockSpec(memory_space=pl.ANY),
                  pl.BlockSpec(memory_space=pltpu.MemorySpace.VMEM)],
        out_specs=pl.BlockSpec(memory_space=pl.ANY),
        scratch_shapes=[
            pltpu.VMEM((2, M, OW), jnp.float32),
            pltpu.VMEM((2, M, OW), jnp.float32),
            pltpu.VMEM((M, OW), jnp.bfloat16),
            pltpu.VMEM((M, OW), jnp.bfloat16),
            pltpu.VMEM((N_DEV - 1, M, OW), jnp.bfloat16),
            pltpu.VMEM((N_DEV - 1, M, OW), jnp.bfloat16),
            pltpu.VMEM((M, PW), jnp.bfloat16),
            pltpu.VMEM((N_DEV, M, PW), jnp.bfloat16),
            pltpu.VMEM((N_DEV, M, PW), jnp.bfloat16),
            pltpu.VMEM((M, D), jnp.float32),
            pltpu.SemaphoreType.DMA((2,)),
            pltpu.SemaphoreType.DMA((2,)),
            pltpu.SemaphoreType.DMA(()),
            pltpu.SemaphoreType.DMA((2,)),
            pltpu.SemaphoreType.DMA((N_DEV - 1,)),
            pltpu.SemaphoreType.DMA((N_DEV - 1,)),
            pltpu.SemaphoreType.DMA((N_DEV - 1,)),
            pltpu.SemaphoreType.DMA((N_DEV - 1,)),
            pltpu.SemaphoreType.DMA((N_DEV - 1,)),
            pltpu.SemaphoreType.DMA((N_DEV,)),
            pltpu.SemaphoreType.DMA((N_DEV,)),
            pltpu.SemaphoreType.DMA((N_DEV,)),
        ],
        compiler_params=pltpu.CompilerParams(
            collective_id=0, vmem_limit_bytes=100 * 1024 * 1024),
    )(x, g2)


# device time: 130289 ns/iter; 2.5138x vs baseline; 1.2444x over previous
import jax
import jax.numpy as jnp
from jax import lax
from jax.experimental import pallas as pl
from jax.experimental.pallas import tpu as pltpu

N_DEV = 4
M = 2048
D = 2048
HW = D // 2
PW = HW // N_DEV
OW = PW // 2
BLK = 512
F32 = jnp.float32
BF16 = jnp.bfloat16


def kernel(partial, gamma):
    x = partial.reshape(N_DEV * M, D)
    mx_out = lax.axis_index("x")
    g_own = lax.dynamic_slice(gamma, (mx_out * HW,), (HW,))
    g_par = lax.dynamic_slice(gamma, ((1 - mx_out) * HW,), (HW,))
    g2 = jnp.concatenate([g_own, g_par]).reshape(1, D)

    def body(x_hbm, g_ref, o_hbm,
             stage_p, stage_m, send_p, send_m, recv_p, recv_m,
             pack_own, buf_y, buf_x, ybuf,
             load_sems_p, load_sems_m, copy_sem, out_sems,
             send_sems_p, recv_sems_p, send_sems_m, recv_sems_m,
             y_send_sems, y_recv_sems, x_send_sems, x_recv_sems):
        z = lax.axis_index("z")
        mx = lax.axis_index("x")
        my = lax.axis_index("y")
        right = lax.rem(z + 1, N_DEV)
        left = lax.rem(z + N_DEV - 1, N_DEV)
        px = 1 - mx
        cbase = mx * HW + my * PW

        def load(idx, col0, dst, sem):
            cp = pltpu.make_async_copy(
                x_hbm.at[pl.ds(idx * M, M), pl.ds(col0, OW)], dst, sem)
            cp.start()
            return cp

        def idx_pair(s):
            if s == N_DEV - 1:
                return z, z
            return (lax.rem(z - 1 - s + 2 * N_DEV, N_DEV),
                    lax.rem(z + 1 + s, N_DEV))

        ip0, im0 = idx_pair(0)
        cps = {0: (load(ip0, cbase, stage_p.at[0], load_sems_p.at[0]),
                   load(im0, cbase + OW, stage_m.at[0], load_sems_m.at[0]))}

        barrier = pltpu.get_barrier_semaphore()
        for dev in [(mx, my, left), (mx, my, right), (px, my, z)]:
            pl.semaphore_signal(barrier, inc=1, device_id=dev,
                                device_id_type=pl.DeviceIdType.MESH)
        for off in range(1, N_DEV):
            pl.semaphore_signal(barrier, inc=1,
                                device_id=(mx, lax.rem(my + off, N_DEV), z),
                                device_id_type=pl.DeviceIdType.MESH)
        pl.semaphore_wait(barrier, 6)

        for s in range(N_DEV - 1):
            slot = s % 2
            nxt = (s + 1) % 2
            cp_p, cp_m = cps.pop(s)
            cp_p.wait()
            cp_m.wait()
            ipn, imn = idx_pair(s + 1)
            cps[s + 1] = (load(ipn, cbase, stage_p.at[nxt],
                               load_sems_p.at[nxt]),
                          load(imn, cbase + OW, stage_m.at[nxt],
                               load_sems_m.at[nxt]))
            for r in range(0, M, BLK):
                rs = slice(r, r + BLK)
                if s == 0:
                    send_p[rs, :] = stage_p[slot, rs, :].astype(BF16)
                    send_m[rs, :] = stage_m[slot, rs, :].astype(BF16)
                else:
                    send_p[rs, :] = (recv_p[s - 1, rs, :].astype(F32)
                                     + stage_p[slot, rs, :]).astype(BF16)
                    send_m[rs, :] = (recv_m[s - 1, rs, :].astype(F32)
                                     + stage_m[slot, rs, :]).astype(BF16)
            rp = pltpu.make_async_remote_copy(
                src_ref=send_p, dst_ref=recv_p.at[s],
                send_sem=send_sems_p.at[s], recv_sem=recv_sems_p.at[s],
                device_id=(mx, my, right),
                device_id_type=pl.DeviceIdType.MESH)
            rm = pltpu.make_async_remote_copy(
                src_ref=send_m, dst_ref=recv_m.at[s],
                send_sem=send_sems_m.at[s], recv_sem=recv_sems_m.at[s],
                device_id=(mx, my, left),
                device_id_type=pl.DeviceIdType.MESH)
            rp.start()
            rm.start()
            rp.wait()
            rm.wait()

        fslot = (N_DEV - 1) % 2
        cp_p, cp_m = cps.pop(N_DEV - 1)
        cp_p.wait()
        cp_m.wait()
        for r in range(0, M, BLK):
            rs = slice(r, r + BLK)
            pack_own[rs, 0:OW] = (recv_p[N_DEV - 2, rs, :].astype(F32)
                                  + stage_p[fslot, rs, :]).astype(BF16)
            pack_own[rs, OW:PW] = (recv_m[N_DEV - 2, rs, :].astype(F32)
                                   + stage_m[fslot, rs, :]).astype(BF16)

        y_rdmas = []
        for off in range(1, N_DEV):
            r_y = pltpu.make_async_remote_copy(
                src_ref=pack_own, dst_ref=buf_y.at[my],
                send_sem=y_send_sems.at[off - 1],
                recv_sem=y_recv_sems.at[off],
                device_id=(mx, lax.rem(my + off, N_DEV), z),
                device_id_type=pl.DeviceIdType.MESH)
            r_y.start()
            y_rdmas.append(r_y)
        x_rdmas = [pltpu.make_async_remote_copy(
            src_ref=pack_own, dst_ref=buf_x.at[my],
            send_sem=x_send_sems.at[0], recv_sem=x_recv_sems.at[0],
            device_id=(px, my, z), device_id_type=pl.DeviceIdType.MESH)]
        x_rdmas[0].start()
        cp_self = pltpu.make_async_copy(pack_own, buf_y.at[my], copy_sem)
        cp_self.start()
        cp_self.wait()
        for off in range(1, N_DEV):
            pltpu.make_async_remote_copy(
                src_ref=pack_own, dst_ref=buf_y.at[0],
                send_sem=y_send_sems.at[off - 1],
                recv_sem=y_recv_sems.at[off],
                device_id=(mx, my, z),
                device_id_type=pl.DeviceIdType.MESH).wait_recv()
            sy = lax.rem(my - off + N_DEV, N_DEV)
            r_x = pltpu.make_async_remote_copy(
                src_ref=buf_y.at[sy], dst_ref=buf_x.at[sy],
                send_sem=x_send_sems.at[off], recv_sem=x_recv_sems.at[off],
                device_id=(px, my, z), device_id_type=pl.DeviceIdType.MESH)
            r_x.start()
            x_rdmas.append(r_x)
        for r_x in x_rdmas:
            r_x.wait_recv()
        for r_y in y_rdmas:
            r_y.wait_send()
        for r_x in x_rdmas:
            r_x.wait_send()

        for r in range(0, M, BLK):
            rs = slice(r, r + BLK)
            ssq = jnp.zeros((BLK, 1), F32)
            for sl in range(N_DEV):
                v = buf_y[sl, rs, :].astype(F32)
                ybuf[rs, sl * PW:(sl + 1) * PW] = v
                ssq = ssq + jnp.sum(v * v, axis=1, keepdims=True)
            for sl in range(N_DEV):
                v = buf_x[sl, rs, :].astype(F32)
                ybuf[rs, HW + sl * PW:HW + (sl + 1) * PW] = v
                ssq = ssq + jnp.sum(v * v, axis=1, keepdims=True)
            inv = lax.rsqrt(ssq / D + 1e-6)
            ybuf[rs, 0:HW] = ybuf[rs, 0:HW] * inv * g_ref[:, 0:HW]
            ybuf[rs, HW:D] = ybuf[rs, HW:D] * inv * g_ref[:, HW:D]
        o_own = pltpu.make_async_copy(
            ybuf.at[:, pl.ds(0, HW)], o_hbm.at[:, pl.ds(mx * HW, HW)],
            out_sems.at[0])
        o_par = pltpu.make_async_copy(
            ybuf.at[:, pl.ds(HW, HW)], o_hbm.at[:, pl.ds(px * HW, HW)],
            out_sems.at[1])
        o_own.start()
        o_par.start()
        o_own.wait()
        o_par.wait()

    return pl.pallas_call(
        body,
        out_shape=jax.ShapeDtypeStruct((M, D), jnp.float32),
        in_specs=[pl.BlockSpec(memory_space=pl.ANY),
                  pl.BlockSpec(memory_space=pltpu.MemorySpace.VMEM)],
        out_specs=pl.BlockSpec(memory_space=pl.ANY),
        scratch_shapes=[
            pltpu.VMEM((2, M, OW), jnp.float32),
            pltpu.VMEM((2, M, OW), jnp.float32),
            pltpu.VMEM((M, OW), jnp.bfloat16),
            pltpu.VMEM((M, OW), jnp.bfloat16),
            pltpu.VMEM((N_DEV - 1, M, OW), jnp.bfloat16),
            pltpu.VMEM((N_DEV - 1, M, OW), jnp.bfloat16),
            pltpu.VMEM((M, PW), jnp.bfloat16),
            pltpu.VMEM((N_DEV, M, PW), jnp.bfloat16),
            pltpu.VMEM((N_DEV, M, PW), jnp.bfloat16),
            pltpu.VMEM((M, D), jnp.float32),
            pltpu.SemaphoreType.DMA((2,)),
            pltpu.SemaphoreType.DMA((2,)),
            pltpu.SemaphoreType.DMA(()),
            pltpu.SemaphoreType.DMA((2,)),
            pltpu.SemaphoreType.DMA((N_DEV - 1,)),
            pltpu.SemaphoreType.DMA((N_DEV - 1,)),
            pltpu.SemaphoreType.DMA((N_DEV - 1,)),
            pltpu.SemaphoreType.DMA((N_DEV - 1,)),
            pltpu.SemaphoreType.DMA((N_DEV - 1,)),
            pltpu.SemaphoreType.DMA((N_DEV,)),
            pltpu.SemaphoreType.DMA((N_DEV,)),
            pltpu.SemaphoreType.DMA((N_DEV,)),
        ],
        compiler_params=pltpu.CompilerParams(
            collective_id=0, vmem_limit_bytes=100 * 1024 * 1024),
    )(x, g2)
